# baseline (speedup 1.0000x reference)
import functools

import jax
import jax.numpy as jnp
from jax import lax
from jax.experimental import pallas as pl
from jax.experimental.pallas import tpu as pltpu
from jax.experimental.pallas import tpu_sc as plsc

D = 32
L = 16
C = 1280


def _rsqrt(s):
    i = plsc.bitcast(s, jnp.int32)
    i = jnp.int32(0x5F3759DF) - (i >> 1)
    y = plsc.bitcast(i, jnp.float32)
    for _ in range(3):
        y = y * (1.5 - 0.5 * s * y * y)
    return y


def _normalize_chunk(rows, nblk):
    def blk(b, _):
        row_ids = b * L + lax.iota(jnp.int32, L)
        ss = jnp.zeros((L,), jnp.float32)
        cols = []
        for j in range(D):
            cj = jnp.full((L,), j, jnp.int32)
            col = plsc.load_gather(rows, [row_ids, cj])
            cols.append(col)
            ss = ss + col * col
        y = _rsqrt(ss)
        scale = 1.0 / jnp.maximum(ss * y, 1e-12)
        for j in range(D):
            cj = jnp.full((L,), j, jnp.int32)
            plsc.store_scatter(rows, [row_ids, cj], cols[j] * scale)
        return 0

    lax.fori_loop(0, nblk, blk, 0)


def _build(B):
    info = plsc.get_sparse_core_info()
    nc, ns = info.num_cores, info.num_subcores
    nw = nc * ns
    per_w = B // nw
    nch = per_w // C
    assert nch % 2 == 0 and nch * C == per_w
    nblk = C // L

    mesh = plsc.VectorSubcoreMesh(core_axis_name="c", subcore_axis_name="s")

    @functools.partial(
        pl.kernel,
        mesh=mesh,
        out_type=jax.ShapeDtypeStruct((B, D), jnp.float32),
        scratch_types=[
            pltpu.VMEM((per_w,), jnp.int32),
            pltpu.VMEM((2, C, D), jnp.float32),
            pltpu.SemaphoreType.DMA,
            pltpu.SemaphoreType.DMA,
            pltpu.SemaphoreType.DMA,
            pltpu.SemaphoreType.DMA,
        ],
        compiler_params=pltpu.CompilerParams(
            use_tc_tiling_on_sc=False, needs_layout_passes=False
        ),
    )
    def k(x_hbm, emb_hbm, out_hbm, idx_v, rows_v, g0, g1, w0, w1):
        wid = lax.axis_index("s") * nc + lax.axis_index("c")
        w_base = wid * per_w
        gsem = (g0, g1)
        wsem = (w0, w1)

        pltpu.sync_copy(x_hbm.at[pl.ds(w_base, per_w)], idx_v)
        pltpu.async_copy(
            emb_hbm.at[idx_v.at[pl.ds(0, C)]], rows_v.at[0], gsem[0]
        )

        def step(i, _):
            for b in range(2):
                ci = 2 * i + b
                cur = rows_v.at[b]
                nxt = ci + 1

                @pl.when(jnp.logical_and(nxt < nch, ci >= 1))
                def _():
                    pltpu.make_async_copy(
                        rows_v.at[1 - b],
                        out_hbm.at[pl.ds(w_base, C)],
                        wsem[1 - b],
                    ).wait()

                @pl.when(nxt < nch)
                def _():
                    pltpu.async_copy(
                        emb_hbm.at[idx_v.at[pl.ds(nxt * C, C)]],
                        rows_v.at[1 - b],
                        gsem[1 - b],
                    )

                pltpu.make_async_copy(
                    emb_hbm.at[pl.ds(0, C)], cur, gsem[b]
                ).wait()
                _normalize_chunk(cur, nblk)
                pltpu.async_copy(
                    cur, out_hbm.at[pl.ds(w_base + ci * C, C)], wsem[b]
                )
            return 0

        lax.fori_loop(0, nch // 2, step, 0)

        for b in range(2):
            pltpu.make_async_copy(
                rows_v.at[b], out_hbm.at[pl.ds(w_base, C)], wsem[b]
            ).wait()

    return k


def kernel(x, embedding):
    B = x.size
    xf = x.reshape(-1).astype(jnp.int32)
    out = _build(B)(xf, embedding)
    return out.reshape(*x.shape, embedding.shape[1])
